# Initial kernel scaffold; baseline (speedup 1.0000x reference)
import functools
import math

import jax
import jax.numpy as jnp
from jax import lax
from jax.experimental import pallas as pl
from jax.experimental.pallas import tpu as pltpu
from jax.experimental.pallas import tpu_sc as plsc

N = 10000
E = 160000
D = 256
H = 128
NS = 16
NC = 2
CH = 125
EPS = E // NS
NCHUNK = EPS // CH
ROWS_PT = N // NS
EPW = E // (NS * NC)
DEG_ROWS = N // 16
BLK = 1000
GRID = N // BLK

_mesh = plsc.VectorSubcoreMesh(core_axis_name="c", subcore_axis_name="s")


@functools.partial(
    pl.kernel,
    out_type=jax.ShapeDtypeStruct((NC, DEG_ROWS, 16), jnp.float32),
    mesh=_mesh,
    scratch_types=[
        pltpu.VMEM((EPW + 16,), jnp.int32),
        pltpu.VMEM((5, CH), jnp.int32),
        pltpu.VMEM((DEG_ROWS, 16), jnp.float32),
        pltpu.VMEM_SHARED((DEG_ROWS, 16), jnp.float32),
    ],
)
def _deg(dst_hbm, zeros_hbm, rowidx_hbm, out_hbm, dstv, rowidx_v, priv,
         shared):
    c = lax.axis_index("c")
    s = lax.axis_index("s")
    wid = s * NC + c
    pltpu.sync_copy(dst_hbm.at[pl.ds(wid * EPW, EPW)], dstv.at[pl.ds(0, EPW)])
    pltpu.sync_copy(rowidx_hbm, rowidx_v)
    pltpu.sync_copy(zeros_hbm, priv)

    @pl.when(s == 0)
    def _zero_shared():
        pltpu.sync_copy(priv, shared)

    plsc.subcore_barrier()

    iota = lax.iota(jnp.int32, 16)
    ones = jnp.full((16,), 1.0, jnp.float32)

    def body(q, carry):
        dv = dstv[pl.ds(q * 16, 16)]
        msk = (iota + q * 16) < EPW
        plsc.addupdate_scatter(priv, [dv // 16, dv % 16], ones, mask=msk)
        return carry

    lax.fori_loop(0, (EPW + 15) // 16, body, 0)

    for k in range(5):
        pltpu.sync_copy(priv.at[pl.ds(k * CH, CH)],
                        shared.at[rowidx_v.at[k]], add=True)

    plsc.subcore_barrier()

    @pl.when(s == 0)
    def _drain():
        pltpu.sync_copy(shared, out_hbm.at[c])


@functools.partial(
    pl.kernel,
    out_type=jax.ShapeDtypeStruct((NC, N, H), jnp.float32),
    mesh=_mesh,
    scratch_types=[
        pltpu.VMEM((NCHUNK, CH), jnp.int32),
        pltpu.VMEM((NCHUNK, CH), jnp.int32),
        pltpu.VMEM((CH, H), jnp.float32),
        pltpu.VMEM_SHARED((N, H), jnp.float32),
        pltpu.SemaphoreType.DMA,
    ],
)
def _agg(gp_hbm, srcm_hbm, dstm_hbm, zeros_hbm, out_hbm,
         srcv, dstv, rows, acc, sem):
    c = lax.axis_index("c")
    s = lax.axis_index("s")
    pltpu.sync_copy(srcm_hbm.at[pl.ds(s * NCHUNK, NCHUNK)], srcv)
    pltpu.sync_copy(dstm_hbm.at[pl.ds(s * NCHUNK, NCHUNK)], dstv)
    pltpu.sync_copy(zeros_hbm, acc.at[pl.ds(s * ROWS_PT, ROWS_PT)])
    plsc.subcore_barrier()

    def body(j, carry):
        pltpu.async_copy(gp_hbm.at[c].at[srcv.at[j]], rows, sem).wait()
        pltpu.sync_copy(rows, acc.at[dstv.at[j]], add=True)
        return carry

    lax.fori_loop(0, NCHUNK, body, 0)
    plsc.subcore_barrier()
    pltpu.sync_copy(acc.at[pl.ds(s * ROWS_PT, ROWS_PT)],
                    out_hbm.at[c, pl.ds(s * ROWS_PT, ROWS_PT)])


def _ln(h, g, b):
    mu = jnp.mean(h, axis=-1, keepdims=True)
    var = jnp.mean((h - mu) ** 2, axis=-1, keepdims=True)
    return (h - mu) * lax.rsqrt(var + 1e-5) * g + b


_DNUM = (((1,), (1,)), ((), ()))


def _stage_a_body(x_ref, deg2_ref, ln1g_ref, ln1b_ref, win_ref, bin_ref,
                  ln2g_ref, ln2b_ref, wg_ref, h_out, gp_out):
    x = x_ref[...]
    deg = deg2_ref[0] + deg2_ref[1] + 1.0
    dinv = lax.rsqrt(deg)
    h = _ln(x, ln1g_ref[...], ln1b_ref[...])
    h = lax.dot_general(h, win_ref[...], _DNUM,
                        preferred_element_type=jnp.float32) + bin_ref[...]
    h = 0.5 * h * (1.0 + lax.erf(h * (1.0 / math.sqrt(2.0))))
    g = _ln(h, ln2g_ref[...], ln2b_ref[...])
    g = lax.dot_general(g, wg_ref[...], _DNUM,
                        preferred_element_type=jnp.float32)
    gp = dinv * g
    h_out[...] = h
    gp_out[0] = gp[:, :H]
    gp_out[1] = gp[:, H:]


def _stage_a(x, deg2, ln1g, ln1b, win, b_in, ln2g, ln2b, wg):
    return pl.pallas_call(
        _stage_a_body,
        grid=(GRID,),
        in_specs=[
            pl.BlockSpec((BLK, D), lambda i: (i, 0)),
            pl.BlockSpec((NC, BLK, 1), lambda i: (0, i, 0)),
            pl.BlockSpec((1, D), lambda i: (0, 0)),
            pl.BlockSpec((1, D), lambda i: (0, 0)),
            pl.BlockSpec((D, D), lambda i: (0, 0)),
            pl.BlockSpec((1, D), lambda i: (0, 0)),
            pl.BlockSpec((1, D), lambda i: (0, 0)),
            pl.BlockSpec((1, D), lambda i: (0, 0)),
            pl.BlockSpec((D, D), lambda i: (0, 0)),
        ],
        out_specs=[
            pl.BlockSpec((BLK, D), lambda i: (i, 0)),
            pl.BlockSpec((NC, BLK, H), lambda i: (0, i, 0)),
        ],
        out_shape=[
            jax.ShapeDtypeStruct((N, D), jnp.float32),
            jax.ShapeDtypeStruct((NC, N, H), jnp.float32),
        ],
    )(x, deg2, ln1g, ln1b, win, b_in, ln2g, ln2b, wg)


def _stage_b_body(x_ref, h_ref, gp_ref, agg_ref, deg2_ref, bg_ref, wout_ref,
                  bout_ref, out_ref):
    deg = deg2_ref[0] + deg2_ref[1] + 1.0
    dinv = lax.rsqrt(deg)
    aggf = jnp.concatenate(
        [agg_ref[0] + gp_ref[0], agg_ref[1] + gp_ref[1]], axis=1)
    gate = jnp.tanh(dinv * aggf + bg_ref[...])
    m = gate * h_ref[...]
    out_ref[...] = x_ref[...] + lax.dot_general(
        m, wout_ref[...], _DNUM,
        preferred_element_type=jnp.float32) + bout_ref[...]


def _stage_b(x, h, gp, agg, deg2, bg, wout, bout):
    return pl.pallas_call(
        _stage_b_body,
        grid=(GRID,),
        in_specs=[
            pl.BlockSpec((BLK, D), lambda i: (i, 0)),
            pl.BlockSpec((BLK, D), lambda i: (i, 0)),
            pl.BlockSpec((NC, BLK, H), lambda i: (0, i, 0)),
            pl.BlockSpec((NC, BLK, H), lambda i: (0, i, 0)),
            pl.BlockSpec((NC, BLK, 1), lambda i: (0, i, 0)),
            pl.BlockSpec((1, D), lambda i: (0, 0)),
            pl.BlockSpec((D, D), lambda i: (0, 0)),
            pl.BlockSpec((1, D), lambda i: (0, 0)),
        ],
        out_specs=pl.BlockSpec((BLK, D), lambda i: (i, 0)),
        out_shape=jax.ShapeDtypeStruct((N, D), jnp.float32),
    )(x, h, gp, agg, deg2, bg, wout, bout)


@jax.jit
def kernel(x, edge_index, ln1_g, ln1_b, Win, b_in, ln2_g, ln2_b, Wg, bg,
           Wout, bout):
    src = edge_index[0].astype(jnp.int32)
    dst = edge_index[1].astype(jnp.int32)
    srcm = src.reshape(E // CH, CH)
    dstm = dst.reshape(E // CH, CH)
    zeros_deg = jnp.zeros((DEG_ROWS, 16), jnp.float32)
    zeros_agg = jnp.zeros((ROWS_PT, H), jnp.float32)
    rowidx = jnp.arange(DEG_ROWS, dtype=jnp.int32).reshape(5, CH)

    deg2 = _deg(dst, zeros_deg, rowidx)
    deg2 = deg2.reshape(NC, N, 1)

    for i in range(3):
        h, gp = _stage_a(x, deg2, ln1_g[i:i + 1], ln1_b[i:i + 1], Win[i],
                         b_in[i:i + 1], ln2_g[i:i + 1], ln2_b[i:i + 1], Wg[i])
        agg = _agg(gp, srcm, dstm, zeros_agg)
        x = _stage_b(x, h, gp, agg, deg2, bg[i:i + 1], Wout[i],
                     bout[i:i + 1])
    return x


# speedup vs baseline: 7.7703x; 7.7703x over previous
import functools
import math

import jax
import jax.numpy as jnp
from jax import lax
from jax.experimental import pallas as pl
from jax.experimental.pallas import tpu as pltpu
from jax.experimental.pallas import tpu_sc as plsc

N = 10000
E = 160000
D = 256
H = 128
NS = 16
NC = 2
CH = 125
EPS = E // NS
NCHUNK = EPS // CH
NP = 10240
ROWS_PT = NP // NS
EPW = E // (NS * NC)
DEG_ROWS = N // 16
BLK = 1000
GRID = N // BLK

_mesh = plsc.VectorSubcoreMesh(core_axis_name="c", subcore_axis_name="s")


@functools.partial(
    pl.kernel,
    out_type=jax.ShapeDtypeStruct((NC * NS, N), jnp.float32),
    mesh=_mesh,
    scratch_types=[
        pltpu.VMEM((EPW + 16,), jnp.int32),
        pltpu.VMEM((N,), jnp.float32),
    ],
    compiler_params=pltpu.CompilerParams(needs_layout_passes=False),
)
def _deg(dst_hbm, zeros_hbm, out_hbm, dstv, priv):
    c = lax.axis_index("c")
    s = lax.axis_index("s")
    wid = s * NC + c
    pltpu.sync_copy(dst_hbm.at[pl.ds(wid * EPW, EPW)], dstv.at[pl.ds(0, EPW)])
    pltpu.sync_copy(zeros_hbm, priv)

    iota = lax.iota(jnp.int32, 16)
    ones = jnp.full((16,), 1.0, jnp.float32)

    def body(q, carry):
        dv = dstv[pl.ds(q * 16, 16)]
        msk = (iota + q * 16) < EPW
        plsc.addupdate_scatter(priv, [dv], ones, mask=msk)
        return carry

    lax.fori_loop(0, (EPW + 15) // 16, body, 0)
    pltpu.sync_copy(priv, out_hbm.at[wid])


@functools.partial(
    pl.kernel,
    out_type=jax.ShapeDtypeStruct((NC, NP, H), jnp.float32),
    mesh=_mesh,
    scratch_types=[
        pltpu.VMEM((NCHUNK, CH), jnp.int32),
        pltpu.VMEM((NCHUNK, CH), jnp.int32),
        pltpu.VMEM((CH, H), jnp.float32),
        pltpu.VMEM_SHARED((NP, H), jnp.float32),
        pltpu.SemaphoreType.DMA,
    ],
    compiler_params=pltpu.CompilerParams(needs_layout_passes=False),
)
def _agg(gp_hbm, srcm_hbm, dstm_hbm, zeros_hbm, out_hbm,
         srcv, dstv, rows, acc, sem):
    c = lax.axis_index("c")
    s = lax.axis_index("s")
    pltpu.sync_copy(srcm_hbm.at[pl.ds(s * NCHUNK, NCHUNK)], srcv)
    pltpu.sync_copy(dstm_hbm.at[pl.ds(s * NCHUNK, NCHUNK)], dstv)
    pltpu.sync_copy(zeros_hbm, acc.at[pl.ds(s * ROWS_PT, ROWS_PT)])
    plsc.subcore_barrier()

    def body(j, carry):
        pltpu.async_copy(gp_hbm.at[c].at[srcv.at[j]], rows, sem).wait()
        pltpu.sync_copy(rows, acc.at[dstv.at[j]], add=True)
        return carry

    lax.fori_loop(0, NCHUNK, body, 0)
    plsc.subcore_barrier()
    pltpu.sync_copy(acc.at[pl.ds(s * ROWS_PT, ROWS_PT)],
                    out_hbm.at[c, pl.ds(s * ROWS_PT, ROWS_PT)])


def _ln(h, g, b):
    mu = jnp.mean(h, axis=-1, keepdims=True)
    var = jnp.mean((h - mu) ** 2, axis=-1, keepdims=True)
    return (h - mu) * lax.rsqrt(var + 1e-5) * g + b


_DNUM = (((1,), (1,)), ((), ()))


def _stage_a_body(x_ref, deg2_ref, ln1g_ref, ln1b_ref, win_ref, bin_ref,
                  ln2g_ref, ln2b_ref, wg_ref, h_out, gp_out):
    x = x_ref[...]
    deg = jnp.sum(deg2_ref[...], axis=0) + 1.0
    dinv = lax.rsqrt(deg)
    h = _ln(x, ln1g_ref[...], ln1b_ref[...])
    h = lax.dot_general(h, win_ref[...], _DNUM,
                        preferred_element_type=jnp.float32) + bin_ref[...]
    h = 0.5 * h * (1.0 + lax.erf(h * (1.0 / math.sqrt(2.0))))
    g = _ln(h, ln2g_ref[...], ln2b_ref[...])
    g = lax.dot_general(g, wg_ref[...], _DNUM,
                        preferred_element_type=jnp.float32)
    gp = dinv * g
    h_out[...] = h
    gp_out[0] = gp[:, :H]
    gp_out[1] = gp[:, H:]


def _stage_a(x, deg2, ln1g, ln1b, win, b_in, ln2g, ln2b, wg):
    return pl.pallas_call(
        _stage_a_body,
        grid=(GRID,),
        in_specs=[
            pl.BlockSpec((BLK, D), lambda i: (i, 0)),
            pl.BlockSpec((NC * NS, BLK, 1), lambda i: (0, i, 0)),
            pl.BlockSpec((1, D), lambda i: (0, 0)),
            pl.BlockSpec((1, D), lambda i: (0, 0)),
            pl.BlockSpec((D, D), lambda i: (0, 0)),
            pl.BlockSpec((1, D), lambda i: (0, 0)),
            pl.BlockSpec((1, D), lambda i: (0, 0)),
            pl.BlockSpec((1, D), lambda i: (0, 0)),
            pl.BlockSpec((D, D), lambda i: (0, 0)),
        ],
        out_specs=[
            pl.BlockSpec((BLK, D), lambda i: (i, 0)),
            pl.BlockSpec((NC, BLK, H), lambda i: (0, i, 0)),
        ],
        out_shape=[
            jax.ShapeDtypeStruct((N, D), jnp.float32),
            jax.ShapeDtypeStruct((NC, N, H), jnp.float32),
        ],
    )(x, deg2, ln1g, ln1b, win, b_in, ln2g, ln2b, wg)


def _stage_b_body(x_ref, h_ref, gp_ref, agg_ref, deg2_ref, bg_ref, wout_ref,
                  bout_ref, out_ref):
    deg = jnp.sum(deg2_ref[...], axis=0) + 1.0
    dinv = lax.rsqrt(deg)
    aggf = jnp.concatenate(
        [agg_ref[0] + gp_ref[0], agg_ref[1] + gp_ref[1]], axis=1)
    gate = jnp.tanh(dinv * aggf + bg_ref[...])
    m = gate * h_ref[...]
    out_ref[...] = x_ref[...] + lax.dot_general(
        m, wout_ref[...], _DNUM,
        preferred_element_type=jnp.float32) + bout_ref[...]


def _stage_b(x, h, gp, agg, deg2, bg, wout, bout):
    return pl.pallas_call(
        _stage_b_body,
        grid=(GRID,),
        in_specs=[
            pl.BlockSpec((BLK, D), lambda i: (i, 0)),
            pl.BlockSpec((BLK, D), lambda i: (i, 0)),
            pl.BlockSpec((NC, BLK, H), lambda i: (0, i, 0)),
            pl.BlockSpec((NC, BLK, H), lambda i: (0, i, 0)),
            pl.BlockSpec((NC * NS, BLK, 1), lambda i: (0, i, 0)),
            pl.BlockSpec((1, D), lambda i: (0, 0)),
            pl.BlockSpec((D, D), lambda i: (0, 0)),
            pl.BlockSpec((1, D), lambda i: (0, 0)),
        ],
        out_specs=pl.BlockSpec((BLK, D), lambda i: (i, 0)),
        out_shape=jax.ShapeDtypeStruct((N, D), jnp.float32),
    )(x, h, gp, agg, deg2, bg, wout, bout)


@jax.jit
def kernel(x, edge_index, ln1_g, ln1_b, Win, b_in, ln2_g, ln2_b, Wg, bg,
           Wout, bout):
    src = edge_index[0].astype(jnp.int32)
    dst = edge_index[1].astype(jnp.int32)
    srcm = src.reshape(E // CH, CH)
    dstm = dst.reshape(E // CH, CH)
    zeros_deg = jnp.zeros((N,), jnp.float32)
    zeros_agg = jnp.zeros((ROWS_PT, H), jnp.float32)

    deg2 = _deg(dst, zeros_deg)
    deg2 = deg2.reshape(NC * NS, N, 1)

    for i in range(3):
        h, gp = _stage_a(x, deg2, ln1_g[i:i + 1], ln1_b[i:i + 1], Win[i],
                         b_in[i:i + 1], ln2_g[i:i + 1], ln2_b[i:i + 1], Wg[i])
        agg = _agg(gp, srcm, dstm, zeros_agg)
        x = _stage_b(x, h, gp, agg, deg2, bg[i:i + 1], Wout[i],
                     bout[i:i + 1])
    return x
